# baseline (speedup 1.0000x reference)
import functools

import jax
import jax.numpy as jnp
from jax import lax
from jax.experimental import pallas as pl
from jax.experimental.pallas import tpu as pltpu
from jax.experimental.pallas import tpu_sc as plsc

N = 10000
E = 320000
F_IN = 128
H = 32
C = 40

NW = 32
CH = 128
K = 80
E_PAD = NW * K * CH
NBUF = 4
N_PAD = 10112
ROWS_PER_TILE = N_PAD // 16


def _segsum_body(y_hbm, src_hbm, dst_hbm, zrow_hbm, out_hbm,
                 src_v, dst_v, rows_v, acc_sh,
                 g0, g1, g2, g3, s0, s1, s2, s3):
    c = lax.axis_index("c")
    s = lax.axis_index("s")
    w = s * 2 + c
    r0 = s * ROWS_PER_TILE
    gsem = [g0, g1, g2, g3]
    ssem = [s0, s1, s2, s3]
    pltpu.sync_copy(zrow_hbm, acc_sh.at[pl.ds(r0, ROWS_PER_TILE)])
    plsc.subcore_barrier()
    pltpu.sync_copy(src_hbm.at[w], src_v)
    pltpu.sync_copy(dst_hbm.at[w], dst_v)

    def gather(j, b):
        pltpu.async_copy(y_hbm.at[src_v.at[j]], rows_v.at[b], gsem[b])

    def wait_gather(j, b):
        pltpu.make_async_copy(y_hbm.at[src_v.at[j]], rows_v.at[b],
                              gsem[b]).wait()

    def scatter(j, b):
        pltpu.async_copy(rows_v.at[b], acc_sh.at[dst_v.at[j]], ssem[b],
                         add=True)

    def wait_scatter(j, b):
        pltpu.make_async_copy(rows_v.at[b], acc_sh.at[dst_v.at[j]],
                              ssem[b]).wait()

    for b in range(NBUF):
        gather(b, b)

    def body(t, carry):
        for b in range(NBUF):
            j = t * NBUF + b
            wait_gather(j, b)
            scatter(j, b)
        for b in range(NBUF):
            j = t * NBUF + b
            wait_scatter(j, b)
            gather(j + NBUF, b)
        return carry

    lax.fori_loop(0, K // NBUF - 1, body, 0)
    for b in range(NBUF):
        j = K - NBUF + b
        wait_gather(j, b)
        scatter(j, b)
    for b in range(NBUF):
        wait_scatter(K - NBUF + b, b)

    plsc.subcore_barrier()
    pltpu.sync_copy(acc_sh.at[pl.ds(r0, ROWS_PER_TILE)],
                    out_hbm.at[c, pl.ds(r0, ROWS_PER_TILE)])


_segsum = functools.partial(
    pl.kernel,
    mesh=plsc.VectorSubcoreMesh(core_axis_name="c", subcore_axis_name="s"),
    out_type=jax.ShapeDtypeStruct((2, N_PAD, H), jnp.float32),
    scratch_types=[
        pltpu.VMEM((K, CH), jnp.int32),
        pltpu.VMEM((K, CH), jnp.int32),
        pltpu.VMEM((NBUF, CH, H), jnp.float32),
        pltpu.VMEM_SHARED((N_PAD, H), jnp.float32),
        pltpu.SemaphoreType.DMA,
        pltpu.SemaphoreType.DMA,
        pltpu.SemaphoreType.DMA,
        pltpu.SemaphoreType.DMA,
        pltpu.SemaphoreType.DMA,
        pltpu.SemaphoreType.DMA,
        pltpu.SemaphoreType.DMA,
        pltpu.SemaphoreType.DMA,
    ],
    compiler_params=pltpu.CompilerParams(use_tc_tiling_on_sc=False),
)(_segsum_body)


def _proj_body(x_ref, w_ref, o_ref):
    o_ref[...] = jnp.dot(x_ref[...], w_ref[...],
                         preferred_element_type=jnp.float32)


def _mid_body(y_ref, ap_ref, b1_ref, w2_ref, b2_ref, g_ref, bb_ref, wn_ref,
              o_ref):
    a = ap_ref[0, :N, :] + ap_ref[1, :N, :]
    t = jnp.maximum(y_ref[...] + a + b1_ref[...], 0.0)
    h = jnp.maximum(jnp.dot(t, w2_ref[...],
                            preferred_element_type=jnp.float32) + b2_ref[...],
                    0.0)
    m = jnp.mean(h, axis=0, keepdims=True)
    d = h - m
    v = jnp.mean(d * d, axis=0, keepdims=True)
    hbn = d * lax.rsqrt(v + 1e-5) * g_ref[...] + bb_ref[...]
    o_ref[...] = jnp.dot(hbn, wn_ref[...], preferred_element_type=jnp.float32)


def _final_body(y_ref, ap_ref, b1_ref, w2_ref, b2_ref, g_ref, bb_ref,
                fc1w_ref, fc1b_ref, fc2w_ref, fc2b_ref, o_ref):
    a = ap_ref[0, :N, :] + ap_ref[1, :N, :]
    t = jnp.maximum(y_ref[...] + a + b1_ref[...], 0.0)
    h = jnp.maximum(jnp.dot(t, w2_ref[...],
                            preferred_element_type=jnp.float32) + b2_ref[...],
                    0.0)
    m = jnp.mean(h, axis=0, keepdims=True)
    d = h - m
    v = jnp.mean(d * d, axis=0, keepdims=True)
    hbn = d * lax.rsqrt(v + 1e-5) * g_ref[...] + bb_ref[...]
    u = jnp.maximum(jnp.dot(hbn, fc1w_ref[...],
                            preferred_element_type=jnp.float32) + fc1b_ref[...],
                    0.0)
    logits = jnp.dot(u, fc2w_ref[...],
                     preferred_element_type=jnp.float32) + fc2b_ref[...]
    mx = jnp.max(logits, axis=-1, keepdims=True)
    e = jnp.exp(logits - mx)
    lse = jnp.log(jnp.sum(e, axis=-1, keepdims=True)) + mx
    o_ref[...] = logits - lse


_proj = pl.pallas_call(
    _proj_body, out_shape=jax.ShapeDtypeStruct((N, H), jnp.float32))
_mid = pl.pallas_call(
    _mid_body, out_shape=jax.ShapeDtypeStruct((N, H), jnp.float32))
_final = pl.pallas_call(
    _final_body, out_shape=jax.ShapeDtypeStruct((N, C), jnp.float32))


def kernel(x, edge_index, w0_1, b0_1, w0_2, b0_2, w1_1, b1_1, w1_2, b1_2,
           bn0_g, bn0_b, bn1_g, bn1_b, fc1_w, fc1_b, fc2_w, fc2_b):
    src = edge_index[0]
    dst = edge_index[1]
    src_p = jnp.pad(src, (0, E_PAD - E)).reshape(NW, K, CH)
    dummy = N + (jnp.arange(E_PAD - E, dtype=jnp.int32) % (N_PAD - N))
    dst_p = jnp.concatenate([dst, dummy]).reshape(NW, K, CH)
    zrow = jnp.zeros((ROWS_PER_TILE, H), jnp.float32)

    b0_1r = b0_1.reshape(1, H)
    b0_2r = b0_2.reshape(1, H)
    b1_1r = b1_1.reshape(1, H)
    b1_2r = b1_2.reshape(1, H)
    g0 = bn0_g.reshape(1, H)
    bb0 = bn0_b.reshape(1, H)
    g1 = bn1_g.reshape(1, H)
    bb1 = bn1_b.reshape(1, H)
    fc1b = fc1_b.reshape(1, H)
    fc2b = fc2_b.reshape(1, C)

    y0 = _proj(x, w0_1)
    a0p = _segsum(y0, src_p, dst_p, zrow)
    y1 = _mid(y0, a0p, b0_1r, w0_2, b0_2r, g0, bb0, w1_1)
    a1p = _segsum(y1, src_p, dst_p, zrow)
    out = _final(y1, a1p, b1_1r, w1_2, b1_2r, g1, bb1,
                 fc1_w, fc1b, fc2_w, fc2b)
    return out


# speedup vs baseline: 1.0545x; 1.0545x over previous
import functools

import jax
import jax.numpy as jnp
from jax import lax
from jax.experimental import pallas as pl
from jax.experimental.pallas import tpu as pltpu
from jax.experimental.pallas import tpu_sc as plsc

N = 10000
E = 320000
F_IN = 128
H = 32
C = 40

NW = 32
CH = 128
K0 = 104
K1 = 56
K_MAX = max(K0, K1)
TOT_CH = 16 * (K0 + K1) + (K_MAX - min(K0, K1))
E_PAD = TOT_CH * CH
NBUF = 4
N_PAD = 10112
ROWS_PER_TILE = N_PAD // 16


def _segsum_body(y_hbm, src_hbm, dst_hbm, zrow_hbm, out_hbm,
                 src_v, dst_v, rows_v, acc_sh,
                 g0, g1, g2, g3, s0, s1, s2, s3):
    c = lax.axis_index("c")
    s = lax.axis_index("s")
    r0 = s * ROWS_PER_TILE
    base = jnp.where(c == 0, s * K0, 16 * K0 + s * K1)
    kc = jnp.where(c == 0, K0, K1)
    gsem = [g0, g1, g2, g3]
    ssem = [s0, s1, s2, s3]
    pltpu.sync_copy(zrow_hbm, acc_sh.at[pl.ds(r0, ROWS_PER_TILE)])
    plsc.subcore_barrier()
    pltpu.sync_copy(src_hbm.at[pl.ds(base, K_MAX)], src_v)
    pltpu.sync_copy(dst_hbm.at[pl.ds(base, K_MAX)], dst_v)

    def gather(j, b):
        pltpu.async_copy(y_hbm.at[src_v.at[j]], rows_v.at[b], gsem[b])

    def wait_gather(j, b):
        pltpu.make_async_copy(y_hbm.at[src_v.at[j]], rows_v.at[b],
                              gsem[b]).wait()

    def scatter(j, b):
        pltpu.async_copy(rows_v.at[b], acc_sh.at[dst_v.at[j]], ssem[b],
                         add=True)

    def wait_scatter(j, b):
        pltpu.make_async_copy(rows_v.at[b], acc_sh.at[dst_v.at[j]],
                              ssem[b]).wait()

    for b in range(NBUF):
        gather(b, b)

    def body(t, carry):
        for b in range(NBUF):
            j = t * NBUF + b
            wait_gather(j, b)
            scatter(j, b)
            wait_scatter(j, b)
            gather(j + NBUF, b)
        return carry

    lax.fori_loop(0, kc // NBUF - 1, body, 0)
    for b in range(NBUF):
        j = kc - NBUF + b
        wait_gather(j, b)
        scatter(j, b)
        wait_scatter(j, b)

    plsc.subcore_barrier()
    pltpu.sync_copy(acc_sh.at[pl.ds(r0, ROWS_PER_TILE)],
                    out_hbm.at[c, pl.ds(r0, ROWS_PER_TILE)])


_segsum = functools.partial(
    pl.kernel,
    mesh=plsc.VectorSubcoreMesh(core_axis_name="c", subcore_axis_name="s"),
    out_type=jax.ShapeDtypeStruct((2, N_PAD, H), jnp.float32),
    scratch_types=[
        pltpu.VMEM((K_MAX, CH), jnp.int32),
        pltpu.VMEM((K_MAX, CH), jnp.int32),
        pltpu.VMEM((NBUF, CH, H), jnp.float32),
        pltpu.VMEM_SHARED((N_PAD, H), jnp.float32),
        pltpu.SemaphoreType.DMA,
        pltpu.SemaphoreType.DMA,
        pltpu.SemaphoreType.DMA,
        pltpu.SemaphoreType.DMA,
        pltpu.SemaphoreType.DMA,
        pltpu.SemaphoreType.DMA,
        pltpu.SemaphoreType.DMA,
        pltpu.SemaphoreType.DMA,
    ],
    compiler_params=pltpu.CompilerParams(use_tc_tiling_on_sc=False),
)(_segsum_body)


def _proj_body(x_ref, w_ref, o_ref):
    o_ref[...] = jnp.dot(x_ref[...], w_ref[...],
                         preferred_element_type=jnp.float32)


def _mid_body(y_ref, ap_ref, b1_ref, w2_ref, b2_ref, g_ref, bb_ref, wn_ref,
              o_ref):
    a = ap_ref[0, :N, :] + ap_ref[1, :N, :]
    t = jnp.maximum(y_ref[...] + a + b1_ref[...], 0.0)
    h = jnp.maximum(jnp.dot(t, w2_ref[...],
                            preferred_element_type=jnp.float32) + b2_ref[...],
                    0.0)
    m = jnp.mean(h, axis=0, keepdims=True)
    d = h - m
    v = jnp.mean(d * d, axis=0, keepdims=True)
    hbn = d * lax.rsqrt(v + 1e-5) * g_ref[...] + bb_ref[...]
    o_ref[...] = jnp.dot(hbn, wn_ref[...], preferred_element_type=jnp.float32)


def _final_body(y_ref, ap_ref, b1_ref, w2_ref, b2_ref, g_ref, bb_ref,
                fc1w_ref, fc1b_ref, fc2w_ref, fc2b_ref, o_ref):
    a = ap_ref[0, :N, :] + ap_ref[1, :N, :]
    t = jnp.maximum(y_ref[...] + a + b1_ref[...], 0.0)
    h = jnp.maximum(jnp.dot(t, w2_ref[...],
                            preferred_element_type=jnp.float32) + b2_ref[...],
                    0.0)
    m = jnp.mean(h, axis=0, keepdims=True)
    d = h - m
    v = jnp.mean(d * d, axis=0, keepdims=True)
    hbn = d * lax.rsqrt(v + 1e-5) * g_ref[...] + bb_ref[...]
    u = jnp.maximum(jnp.dot(hbn, fc1w_ref[...],
                            preferred_element_type=jnp.float32) + fc1b_ref[...],
                    0.0)
    logits = jnp.dot(u, fc2w_ref[...],
                     preferred_element_type=jnp.float32) + fc2b_ref[...]
    mx = jnp.max(logits, axis=-1, keepdims=True)
    e = jnp.exp(logits - mx)
    lse = jnp.log(jnp.sum(e, axis=-1, keepdims=True)) + mx
    o_ref[...] = logits - lse


_proj = pl.pallas_call(
    _proj_body, out_shape=jax.ShapeDtypeStruct((N, H), jnp.float32))
_mid = pl.pallas_call(
    _mid_body, out_shape=jax.ShapeDtypeStruct((N, H), jnp.float32))
_final = pl.pallas_call(
    _final_body, out_shape=jax.ShapeDtypeStruct((N, C), jnp.float32))


def kernel(x, edge_index, w0_1, b0_1, w0_2, b0_2, w1_1, b1_1, w1_2, b1_2,
           bn0_g, bn0_b, bn1_g, bn1_b, fc1_w, fc1_b, fc2_w, fc2_b):
    src = edge_index[0]
    dst = edge_index[1]
    src_p = jnp.pad(src, (0, E_PAD - E)).reshape(TOT_CH, CH)
    dummy = N + (jnp.arange(E_PAD - E, dtype=jnp.int32) % (N_PAD - N))
    dst_p = jnp.concatenate([dst, dummy]).reshape(TOT_CH, CH)
    zrow = jnp.zeros((ROWS_PER_TILE, H), jnp.float32)

    b0_1r = b0_1.reshape(1, H)
    b0_2r = b0_2.reshape(1, H)
    b1_1r = b1_1.reshape(1, H)
    b1_2r = b1_2.reshape(1, H)
    g0 = bn0_g.reshape(1, H)
    bb0 = bn0_b.reshape(1, H)
    g1 = bn1_g.reshape(1, H)
    bb1 = bn1_b.reshape(1, H)
    fc1b = fc1_b.reshape(1, H)
    fc2b = fc2_b.reshape(1, C)

    y0 = _proj(x, w0_1)
    a0p = _segsum(y0, src_p, dst_p, zrow)
    y1 = _mid(y0, a0p, b0_1r, w0_2, b0_2r, g0, bb0, w1_1)
    a1p = _segsum(y1, src_p, dst_p, zrow)
    out = _final(y1, a1p, b1_1r, w1_2, b1_2r, g1, bb1,
                 fc1_w, fc1b, fc2_w, fc2b)
    return out


# speedup vs baseline: 1.0586x; 1.0039x over previous
import functools

import jax
import jax.numpy as jnp
from jax import lax
from jax.experimental import pallas as pl
from jax.experimental.pallas import tpu as pltpu
from jax.experimental.pallas import tpu_sc as plsc

N = 10000
E = 320000
F_IN = 128
H = 32
C = 40

NW = 32
CH = 128
K0 = 120
K1 = 40
K_MAX = max(K0, K1)
TOT_CH = 16 * (K0 + K1) + (K_MAX - min(K0, K1))
E_PAD = TOT_CH * CH
NBUF = 4
N_PAD = 10112
ROWS_PER_TILE = N_PAD // 16


def _segsum_body(y_hbm, src_hbm, dst_hbm, zrow_hbm, out_hbm,
                 src_v, dst_v, rows_v, acc_sh,
                 g0, g1, g2, g3, s0, s1, s2, s3):
    c = lax.axis_index("c")
    s = lax.axis_index("s")
    r0 = s * ROWS_PER_TILE
    base = jnp.where(c == 0, s * K0, 16 * K0 + s * K1)
    kc = jnp.where(c == 0, K0, K1)
    gsem = [g0, g1, g2, g3]
    ssem = [s0, s1, s2, s3]
    pltpu.sync_copy(zrow_hbm, acc_sh.at[pl.ds(r0, ROWS_PER_TILE)])
    plsc.subcore_barrier()
    pltpu.sync_copy(src_hbm.at[pl.ds(base, K_MAX)], src_v)
    pltpu.sync_copy(dst_hbm.at[pl.ds(base, K_MAX)], dst_v)

    def gather(j, b):
        pltpu.async_copy(y_hbm.at[src_v.at[j]], rows_v.at[b], gsem[b])

    def wait_gather(j, b):
        pltpu.make_async_copy(y_hbm.at[src_v.at[j]], rows_v.at[b],
                              gsem[b]).wait()

    def scatter(j, b):
        pltpu.async_copy(rows_v.at[b], acc_sh.at[dst_v.at[j]], ssem[b],
                         add=True)

    def wait_scatter(j, b):
        pltpu.make_async_copy(rows_v.at[b], acc_sh.at[dst_v.at[j]],
                              ssem[b]).wait()

    for b in range(NBUF):
        gather(b, b)

    def body(t, carry):
        for b in range(NBUF):
            j = t * NBUF + b
            wait_gather(j, b)
            scatter(j, b)
            wait_scatter(j, b)
            gather(j + NBUF, b)
        return carry

    lax.fori_loop(0, kc // NBUF - 1, body, 0)
    for b in range(NBUF):
        j = kc - NBUF + b
        wait_gather(j, b)
        scatter(j, b)
        wait_scatter(j, b)

    plsc.subcore_barrier()
    pltpu.sync_copy(acc_sh.at[pl.ds(r0, ROWS_PER_TILE)],
                    out_hbm.at[c, pl.ds(r0, ROWS_PER_TILE)])


_segsum = functools.partial(
    pl.kernel,
    mesh=plsc.VectorSubcoreMesh(core_axis_name="c", subcore_axis_name="s"),
    out_type=jax.ShapeDtypeStruct((2, N_PAD, H), jnp.float32),
    scratch_types=[
        pltpu.VMEM((K_MAX, CH), jnp.int32),
        pltpu.VMEM((K_MAX, CH), jnp.int32),
        pltpu.VMEM((NBUF, CH, H), jnp.float32),
        pltpu.VMEM_SHARED((N_PAD, H), jnp.float32),
        pltpu.SemaphoreType.DMA,
        pltpu.SemaphoreType.DMA,
        pltpu.SemaphoreType.DMA,
        pltpu.SemaphoreType.DMA,
        pltpu.SemaphoreType.DMA,
        pltpu.SemaphoreType.DMA,
        pltpu.SemaphoreType.DMA,
        pltpu.SemaphoreType.DMA,
    ],
    compiler_params=pltpu.CompilerParams(use_tc_tiling_on_sc=False),
)(_segsum_body)


def _proj_body(x_ref, w_ref, o_ref):
    o_ref[...] = jnp.dot(x_ref[...], w_ref[...],
                         preferred_element_type=jnp.float32)


def _mid_body(y_ref, ap_ref, b1_ref, w2_ref, b2_ref, g_ref, bb_ref, wn_ref,
              o_ref):
    a = ap_ref[0, :N, :] + ap_ref[1, :N, :]
    t = jnp.maximum(y_ref[...] + a + b1_ref[...], 0.0)
    h = jnp.maximum(jnp.dot(t, w2_ref[...],
                            preferred_element_type=jnp.float32) + b2_ref[...],
                    0.0)
    m = jnp.mean(h, axis=0, keepdims=True)
    d = h - m
    v = jnp.mean(d * d, axis=0, keepdims=True)
    hbn = d * lax.rsqrt(v + 1e-5) * g_ref[...] + bb_ref[...]
    o_ref[...] = jnp.dot(hbn, wn_ref[...], preferred_element_type=jnp.float32)


def _final_body(y_ref, ap_ref, b1_ref, w2_ref, b2_ref, g_ref, bb_ref,
                fc1w_ref, fc1b_ref, fc2w_ref, fc2b_ref, o_ref):
    a = ap_ref[0, :N, :] + ap_ref[1, :N, :]
    t = jnp.maximum(y_ref[...] + a + b1_ref[...], 0.0)
    h = jnp.maximum(jnp.dot(t, w2_ref[...],
                            preferred_element_type=jnp.float32) + b2_ref[...],
                    0.0)
    m = jnp.mean(h, axis=0, keepdims=True)
    d = h - m
    v = jnp.mean(d * d, axis=0, keepdims=True)
    hbn = d * lax.rsqrt(v + 1e-5) * g_ref[...] + bb_ref[...]
    u = jnp.maximum(jnp.dot(hbn, fc1w_ref[...],
                            preferred_element_type=jnp.float32) + fc1b_ref[...],
                    0.0)
    logits = jnp.dot(u, fc2w_ref[...],
                     preferred_element_type=jnp.float32) + fc2b_ref[...]
    mx = jnp.max(logits, axis=-1, keepdims=True)
    e = jnp.exp(logits - mx)
    lse = jnp.log(jnp.sum(e, axis=-1, keepdims=True)) + mx
    o_ref[...] = logits - lse


_proj = pl.pallas_call(
    _proj_body, out_shape=jax.ShapeDtypeStruct((N, H), jnp.float32))
_mid = pl.pallas_call(
    _mid_body, out_shape=jax.ShapeDtypeStruct((N, H), jnp.float32))
_final = pl.pallas_call(
    _final_body, out_shape=jax.ShapeDtypeStruct((N, C), jnp.float32))


def kernel(x, edge_index, w0_1, b0_1, w0_2, b0_2, w1_1, b1_1, w1_2, b1_2,
           bn0_g, bn0_b, bn1_g, bn1_b, fc1_w, fc1_b, fc2_w, fc2_b):
    src = edge_index[0]
    dst = edge_index[1]
    src_p = jnp.pad(src, (0, E_PAD - E)).reshape(TOT_CH, CH)
    dummy = N + (jnp.arange(E_PAD - E, dtype=jnp.int32) % (N_PAD - N))
    dst_p = jnp.concatenate([dst, dummy]).reshape(TOT_CH, CH)
    zrow = jnp.zeros((ROWS_PER_TILE, H), jnp.float32)

    b0_1r = b0_1.reshape(1, H)
    b0_2r = b0_2.reshape(1, H)
    b1_1r = b1_1.reshape(1, H)
    b1_2r = b1_2.reshape(1, H)
    g0 = bn0_g.reshape(1, H)
    bb0 = bn0_b.reshape(1, H)
    g1 = bn1_g.reshape(1, H)
    bb1 = bn1_b.reshape(1, H)
    fc1b = fc1_b.reshape(1, H)
    fc2b = fc2_b.reshape(1, C)

    y0 = _proj(x, w0_1)
    a0p = _segsum(y0, src_p, dst_p, zrow)
    y1 = _mid(y0, a0p, b0_1r, w0_2, b0_2r, g0, bb0, w1_1)
    a1p = _segsum(y1, src_p, dst_p, zrow)
    out = _final(y1, a1p, b1_1r, w1_2, b1_2r, g1, bb1,
                 fc1_w, fc1b, fc2_w, fc2b)
    return out


# speedup vs baseline: 1.0819x; 1.0220x over previous
import functools

import jax
import jax.numpy as jnp
from jax import lax
from jax.experimental import pallas as pl
from jax.experimental.pallas import tpu as pltpu
from jax.experimental.pallas import tpu_sc as plsc

N = 10000
E = 320000
F_IN = 128
H = 32
C = 40

NW = 32
CH = 128
K0 = 120
K1 = 40
K_MAX = max(K0, K1)
TOT_CH = 16 * (K0 + K1)
E_PAD = TOT_CH * CH
NBUF = 4
N_PAD = 10112
ROWS_PER_TILE = N_PAD // 16


def _segsum_body(y_hbm, src_hbm, dst_hbm, out_hbm,
                 src_v, dst_v, rows_v, zbuf, acc_sh,
                 g0, g1, g2, g3, s0, s1, s2, s3):
    c = lax.axis_index("c")
    s = lax.axis_index("s")
    r0 = s * ROWS_PER_TILE
    kc = jnp.where(c == 0, K0, K1)
    gsem = [g0, g1, g2, g3]
    ssem = [s0, s1, s2, s3]
    z16 = jnp.zeros((16,), jnp.float32)
    for r in range(CH):
        zbuf[r, pl.ds(0, 16)] = z16
        zbuf[r, pl.ds(16, 16)] = z16
    for kz in range(ROWS_PER_TILE // CH):
        pltpu.sync_copy(zbuf, acc_sh.at[pl.ds(r0 + kz * CH, CH)])
    _ZT = ROWS_PER_TILE % CH
    pltpu.sync_copy(zbuf.at[pl.ds(0, _ZT)],
                    acc_sh.at[pl.ds(r0 + ROWS_PER_TILE - _ZT, _ZT)])
    plsc.subcore_barrier()
    @pl.when(c == 0)
    def _stage0():
        pltpu.sync_copy(src_hbm.at[pl.ds(s * K0, K0)],
                        src_v.at[pl.ds(0, K0)])
        pltpu.sync_copy(dst_hbm.at[pl.ds(s * K0, K0)],
                        dst_v.at[pl.ds(0, K0)])

    @pl.when(c == 1)
    def _stage1():
        pltpu.sync_copy(src_hbm.at[pl.ds(16 * K0 + s * K1, K1)],
                        src_v.at[pl.ds(0, K1)])
        pltpu.sync_copy(dst_hbm.at[pl.ds(16 * K0 + s * K1, K1)],
                        dst_v.at[pl.ds(0, K1)])

    def gather(j, b):
        pltpu.async_copy(y_hbm.at[src_v.at[j]], rows_v.at[b], gsem[b])

    def wait_gather(j, b):
        pltpu.make_async_copy(y_hbm.at[src_v.at[j]], rows_v.at[b],
                              gsem[b]).wait()

    def scatter(j, b):
        pltpu.async_copy(rows_v.at[b], acc_sh.at[dst_v.at[j]], ssem[b],
                         add=True)

    def wait_scatter(j, b):
        pltpu.make_async_copy(rows_v.at[b], acc_sh.at[dst_v.at[j]],
                              ssem[b]).wait()

    for b in range(NBUF):
        gather(b, b)

    def body(t, carry):
        for b in range(NBUF):
            j = t * NBUF + b
            wait_gather(j, b)
            scatter(j, b)
            wait_scatter(j, b)
            gather(j + NBUF, b)
        return carry

    lax.fori_loop(0, kc // NBUF - 1, body, 0)
    for b in range(NBUF):
        j = kc - NBUF + b
        wait_gather(j, b)
        scatter(j, b)
        wait_scatter(j, b)

    plsc.subcore_barrier()
    pltpu.sync_copy(acc_sh.at[pl.ds(r0, ROWS_PER_TILE)],
                    out_hbm.at[c, pl.ds(r0, ROWS_PER_TILE)])


_segsum = functools.partial(
    pl.kernel,
    mesh=plsc.VectorSubcoreMesh(core_axis_name="c", subcore_axis_name="s"),
    out_type=jax.ShapeDtypeStruct((2, N_PAD, H), jnp.float32),
    scratch_types=[
        pltpu.VMEM((K_MAX, CH), jnp.int32),
        pltpu.VMEM((K_MAX, CH), jnp.int32),
        pltpu.VMEM((NBUF, CH, H), jnp.float32),
        pltpu.VMEM((CH, H), jnp.float32),
        pltpu.VMEM_SHARED((N_PAD, H), jnp.float32),
        pltpu.SemaphoreType.DMA,
        pltpu.SemaphoreType.DMA,
        pltpu.SemaphoreType.DMA,
        pltpu.SemaphoreType.DMA,
        pltpu.SemaphoreType.DMA,
        pltpu.SemaphoreType.DMA,
        pltpu.SemaphoreType.DMA,
        pltpu.SemaphoreType.DMA,
    ],
    compiler_params=pltpu.CompilerParams(use_tc_tiling_on_sc=False),
)(_segsum_body)


def _proj_body(x_ref, w_ref, o_ref):
    o_ref[...] = jnp.dot(x_ref[...], w_ref[...],
                         preferred_element_type=jnp.float32)


def _mid_body(y_ref, ap_ref, b1_ref, w2_ref, b2_ref, g_ref, bb_ref, wn_ref,
              o_ref):
    a = ap_ref[0, :N, :] + ap_ref[1, :N, :]
    t = jnp.maximum(y_ref[...] + a + b1_ref[...], 0.0)
    h = jnp.maximum(jnp.dot(t, w2_ref[...],
                            preferred_element_type=jnp.float32) + b2_ref[...],
                    0.0)
    m = jnp.mean(h, axis=0, keepdims=True)
    d = h - m
    v = jnp.mean(d * d, axis=0, keepdims=True)
    hbn = d * lax.rsqrt(v + 1e-5) * g_ref[...] + bb_ref[...]
    o_ref[...] = jnp.dot(hbn, wn_ref[...], preferred_element_type=jnp.float32)


def _final_body(y_ref, ap_ref, b1_ref, w2_ref, b2_ref, g_ref, bb_ref,
                fc1w_ref, fc1b_ref, fc2w_ref, fc2b_ref, o_ref):
    a = ap_ref[0, :N, :] + ap_ref[1, :N, :]
    t = jnp.maximum(y_ref[...] + a + b1_ref[...], 0.0)
    h = jnp.maximum(jnp.dot(t, w2_ref[...],
                            preferred_element_type=jnp.float32) + b2_ref[...],
                    0.0)
    m = jnp.mean(h, axis=0, keepdims=True)
    d = h - m
    v = jnp.mean(d * d, axis=0, keepdims=True)
    hbn = d * lax.rsqrt(v + 1e-5) * g_ref[...] + bb_ref[...]
    u = jnp.maximum(jnp.dot(hbn, fc1w_ref[...],
                            preferred_element_type=jnp.float32) + fc1b_ref[...],
                    0.0)
    logits = jnp.dot(u, fc2w_ref[...],
                     preferred_element_type=jnp.float32) + fc2b_ref[...]
    mx = jnp.max(logits, axis=-1, keepdims=True)
    e = jnp.exp(logits - mx)
    lse = jnp.log(jnp.sum(e, axis=-1, keepdims=True)) + mx
    o_ref[...] = logits - lse


_proj = pl.pallas_call(
    _proj_body, out_shape=jax.ShapeDtypeStruct((N, H), jnp.float32))
_mid = pl.pallas_call(
    _mid_body, out_shape=jax.ShapeDtypeStruct((N, H), jnp.float32))
_final = pl.pallas_call(
    _final_body, out_shape=jax.ShapeDtypeStruct((N, C), jnp.float32))


def kernel(x, edge_index, w0_1, b0_1, w0_2, b0_2, w1_1, b1_1, w1_2, b1_2,
           bn0_g, bn0_b, bn1_g, bn1_b, fc1_w, fc1_b, fc2_w, fc2_b):
    src = edge_index[0]
    dst = edge_index[1]
    src_p = jnp.pad(src, (0, E_PAD - E)).reshape(TOT_CH, CH)
    dummy = N + (jnp.arange(E_PAD - E, dtype=jnp.int32) % (N_PAD - N))
    dst_p = jnp.concatenate([dst, dummy]).reshape(TOT_CH, CH)

    b0_1r = b0_1.reshape(1, H)
    b0_2r = b0_2.reshape(1, H)
    b1_1r = b1_1.reshape(1, H)
    b1_2r = b1_2.reshape(1, H)
    g0 = bn0_g.reshape(1, H)
    bb0 = bn0_b.reshape(1, H)
    g1 = bn1_g.reshape(1, H)
    bb1 = bn1_b.reshape(1, H)
    fc1b = fc1_b.reshape(1, H)
    fc2b = fc2_b.reshape(1, C)

    y0 = _proj(x, w0_1)
    a0p = _segsum(y0, src_p, dst_p)
    y1 = _mid(y0, a0p, b0_1r, w0_2, b0_2r, g0, bb0, w1_1)
    a1p = _segsum(y1, src_p, dst_p)
    out = _final(y1, a1p, b1_1r, w1_2, b1_2r, g1, bb1,
                 fc1_w, fc1b, fc2_w, fc2b)
    return out
